# baseline (speedup 1.0000x reference)
import functools

import numpy as np
import jax
from jax import lax
import jax.numpy as jnp
from jax.experimental import pallas as pl
from jax.experimental.pallas import tpu as pltpu
from jax.experimental.pallas import tpu_sc as plsc

_NUM_IMGS = 128
_GLOBAL_SIZE = 32
_SEED = 41

_SLAB = 2048 * 256
_COLS = 1024
_ROWS = _SLAB // _COLS
_CHUNK = 32
_NCHUNK = _ROWS // _CHUNK
_NBUF = 3
_NC = 2

_ROTS = ((13, 15, 26, 6), (17, 29, 16, 24))


def _rotl(x, d):
    return ((x << np.uint32(d)) | (x >> np.uint32(32 - d))).astype(np.uint32)


def _hash2x32(k1, k2, x0, x1):
    x = [x0.astype(np.uint32), x1.astype(np.uint32)]
    ks = [np.uint32(k1), np.uint32(k2),
          np.uint32(k1) ^ np.uint32(k2) ^ np.uint32(0x1BD11BDA)]
    x[0] = (x[0] + ks[0]).astype(np.uint32)
    x[1] = (x[1] + ks[1]).astype(np.uint32)
    for i in range(5):
        for r in _ROTS[i % 2]:
            x[0] = (x[0] + x[1]).astype(np.uint32)
            x[1] = _rotl(x[1], r) ^ x[0]
        x[0] = (x[0] + ks[(i + 1) % 3]).astype(np.uint32)
        x[1] = (x[1] + ks[(i + 2) % 3] + np.uint32(i + 1)).astype(np.uint32)
    return x[0], x[1]


def _random_bits32(key, n):
    b1, b2 = _hash2x32(key[0], key[1],
                       np.zeros(n, np.uint32), np.arange(n, dtype=np.uint32))
    return b1 ^ b2


@functools.lru_cache(maxsize=1)
def _sampled_indices() -> tuple[int, ...]:
    key = np.array([_SEED >> 32, _SEED & 0xFFFFFFFF], dtype=np.uint32)
    b1, b2 = _hash2x32(key[0], key[1],
                       np.zeros(2, np.uint32), np.arange(2, dtype=np.uint32))
    k1, k2 = np.stack([b1, b2], axis=1)
    higher = _random_bits32(k1, _GLOBAL_SIZE)
    lower = _random_bits32(k2, _GLOBAL_SIZE)
    span = np.uint32(_NUM_IMGS)
    mult = np.uint32((((2 ** 16) % int(span)) ** 2) % int(span))
    off = ((higher % span) * mult + (lower % span)).astype(np.uint32)
    off = (off % span).astype(np.int32)
    return tuple(int(v) for v in np.sort(off))


def _tec_body(x_hbm, gidx_hbm, sidx_hbm, out_hbm,
              gidx_v, sidx_v, buf, g0, g1, g2, s0, s1, s2):
    gsem = (g0, g1, g2)
    ssem = (s0, s1, s2)
    cid = lax.axis_index("c")
    sid = lax.axis_index("s")
    w = sid * _NC + cid
    pltpu.sync_copy(gidx_hbm.at[w], gidx_v)
    pltpu.sync_copy(sidx_hbm.at[w], sidx_v)

    def gather(ch, b):
        return pltpu.make_async_copy(
            x_hbm.at[gidx_v.at[ch]], buf.at[b], gsem[b])

    def scatter(ch, b):
        return pltpu.make_async_copy(
            buf.at[b], out_hbm.at[sidx_v.at[ch]], ssem[b])

    for b in range(_NBUF):
        gather(b, b).start()
    for ch in range(_NCHUNK):
        b = ch % _NBUF
        gather(ch, b).wait()
        scatter(ch, b).start()
        nxt = ch + _NBUF
        if nxt < _NCHUNK:
            scatter(ch, b).wait()
            gather(nxt, b).start()
    for ch in range(_NCHUNK - _NBUF, _NCHUNK):
        scatter(ch, ch % _NBUF).wait()


def kernel(x):
    n, r, c = x.shape
    idx = np.asarray(_sampled_indices(), dtype=np.int32)
    base = np.arange(_ROWS, dtype=np.int32).reshape(_NCHUNK, _CHUNK)
    gidx = jnp.asarray(idx[:, None, None] * _ROWS + base[None])
    sidx = jnp.asarray(
        np.arange(_GLOBAL_SIZE, dtype=np.int32)[:, None, None] * _ROWS
        + base[None])

    x2d = x.reshape(n * _ROWS, _COLS)
    out2d = pl.kernel(
        _tec_body,
        out_type=jax.ShapeDtypeStruct((_GLOBAL_SIZE * _ROWS, _COLS), x.dtype),
        mesh=plsc.VectorSubcoreMesh(core_axis_name="c", subcore_axis_name="s"),
        scratch_types=(
            [pltpu.VMEM((_NCHUNK, _CHUNK), jnp.int32),
             pltpu.VMEM((_NCHUNK, _CHUNK), jnp.int32),
             pltpu.VMEM((_NBUF, _CHUNK, _COLS), jnp.float32)]
            + [pltpu.SemaphoreType.DMA] * (2 * _NBUF)
        ),
    )(x2d, gidx, sidx)
    return out2d.reshape(_GLOBAL_SIZE, r, c)


# speedup vs baseline: 1.0494x; 1.0494x over previous
import functools

import numpy as np
import jax
from jax import lax
import jax.numpy as jnp
from jax.experimental import pallas as pl
from jax.experimental.pallas import tpu as pltpu
from jax.experimental.pallas import tpu_sc as plsc

_NUM_IMGS = 128
_GLOBAL_SIZE = 32
_SEED = 41

_SLAB = 2048 * 256
_COLS = 128
_ROWS = _SLAB // _COLS
_CHUNK = 128
_NCHUNK = _ROWS // _CHUNK
_NBUF = 3
_NC = 2

_ROTS = ((13, 15, 26, 6), (17, 29, 16, 24))


def _rotl(x, d):
    return ((x << np.uint32(d)) | (x >> np.uint32(32 - d))).astype(np.uint32)


def _hash2x32(k1, k2, x0, x1):
    x = [x0.astype(np.uint32), x1.astype(np.uint32)]
    ks = [np.uint32(k1), np.uint32(k2),
          np.uint32(k1) ^ np.uint32(k2) ^ np.uint32(0x1BD11BDA)]
    x[0] = (x[0] + ks[0]).astype(np.uint32)
    x[1] = (x[1] + ks[1]).astype(np.uint32)
    for i in range(5):
        for r in _ROTS[i % 2]:
            x[0] = (x[0] + x[1]).astype(np.uint32)
            x[1] = _rotl(x[1], r) ^ x[0]
        x[0] = (x[0] + ks[(i + 1) % 3]).astype(np.uint32)
        x[1] = (x[1] + ks[(i + 2) % 3] + np.uint32(i + 1)).astype(np.uint32)
    return x[0], x[1]


def _random_bits32(key, n):
    b1, b2 = _hash2x32(key[0], key[1],
                       np.zeros(n, np.uint32), np.arange(n, dtype=np.uint32))
    return b1 ^ b2


@functools.lru_cache(maxsize=1)
def _sampled_indices() -> tuple[int, ...]:
    key = np.array([_SEED >> 32, _SEED & 0xFFFFFFFF], dtype=np.uint32)
    b1, b2 = _hash2x32(key[0], key[1],
                       np.zeros(2, np.uint32), np.arange(2, dtype=np.uint32))
    k1, k2 = np.stack([b1, b2], axis=1)
    higher = _random_bits32(k1, _GLOBAL_SIZE)
    lower = _random_bits32(k2, _GLOBAL_SIZE)
    span = np.uint32(_NUM_IMGS)
    mult = np.uint32((((2 ** 16) % int(span)) ** 2) % int(span))
    off = ((higher % span) * mult + (lower % span)).astype(np.uint32)
    off = (off % span).astype(np.int32)
    return tuple(int(v) for v in np.sort(off))


def _tec_body(x_hbm, gidx_hbm, sidx_hbm, out_hbm,
              gidx_v, sidx_v, buf, g0, g1, g2, s0, s1, s2):
    gsem = (g0, g1, g2)
    ssem = (s0, s1, s2)
    cid = lax.axis_index("c")
    sid = lax.axis_index("s")
    w = sid * _NC + cid
    pltpu.sync_copy(gidx_hbm.at[w], gidx_v)
    pltpu.sync_copy(sidx_hbm.at[w], sidx_v)

    def gather(ch, b):
        return pltpu.make_async_copy(
            x_hbm.at[gidx_v.at[ch]], buf.at[b], gsem[b])

    def scatter(ch, b):
        return pltpu.make_async_copy(
            buf.at[b], out_hbm.at[sidx_v.at[ch]], ssem[b])

    for b in range(_NBUF):
        gather(b, b).start()
    for ch in range(_NCHUNK):
        b = ch % _NBUF
        gather(ch, b).wait()
        scatter(ch, b).start()
        nxt = ch + _NBUF
        if nxt < _NCHUNK:
            scatter(ch, b).wait()
            gather(nxt, b).start()
    for ch in range(_NCHUNK - _NBUF, _NCHUNK):
        scatter(ch, ch % _NBUF).wait()


def kernel(x):
    n, r, c = x.shape
    idx = np.asarray(_sampled_indices(), dtype=np.int32)
    base = np.arange(_ROWS, dtype=np.int32).reshape(_NCHUNK, _CHUNK)
    gidx = jnp.asarray(idx[:, None, None] * _ROWS + base[None])
    sidx = jnp.asarray(
        np.arange(_GLOBAL_SIZE, dtype=np.int32)[:, None, None] * _ROWS
        + base[None])

    x2d = x.reshape(n * _ROWS, _COLS)
    out2d = pl.kernel(
        _tec_body,
        out_type=jax.ShapeDtypeStruct((_GLOBAL_SIZE * _ROWS, _COLS), x.dtype),
        mesh=plsc.VectorSubcoreMesh(core_axis_name="c", subcore_axis_name="s"),
        scratch_types=(
            [pltpu.VMEM((_NCHUNK, _CHUNK), jnp.int32),
             pltpu.VMEM((_NCHUNK, _CHUNK), jnp.int32),
             pltpu.VMEM((_NBUF, _CHUNK, _COLS), jnp.float32)]
            + [pltpu.SemaphoreType.DMA] * (2 * _NBUF)
        ),
    )(x2d, gidx, sidx)
    return out2d.reshape(_GLOBAL_SIZE, r, c)


# speedup vs baseline: 6.3683x; 6.0683x over previous
import functools

import numpy as np
import jax
from jax import lax
import jax.numpy as jnp
from jax.experimental import pallas as pl
from jax.experimental.pallas import tpu as pltpu
from jax.experimental.pallas import tpu_sc as plsc

_NUM_IMGS = 128
_GLOBAL_SIZE = 32
_SEED = 41

_SLAB = 2048 * 256
_COLS = 256
_ROWS = _SLAB // _COLS
_CHUNK = 128
_NCHUNK = _ROWS // _CHUNK
_NBUF = 3
_NC = 2

_ROTS = ((13, 15, 26, 6), (17, 29, 16, 24))


def _rotl(x, d):
    return ((x << np.uint32(d)) | (x >> np.uint32(32 - d))).astype(np.uint32)


def _hash2x32(k1, k2, x0, x1):
    x = [x0.astype(np.uint32), x1.astype(np.uint32)]
    ks = [np.uint32(k1), np.uint32(k2),
          np.uint32(k1) ^ np.uint32(k2) ^ np.uint32(0x1BD11BDA)]
    x[0] = (x[0] + ks[0]).astype(np.uint32)
    x[1] = (x[1] + ks[1]).astype(np.uint32)
    for i in range(5):
        for r in _ROTS[i % 2]:
            x[0] = (x[0] + x[1]).astype(np.uint32)
            x[1] = _rotl(x[1], r) ^ x[0]
        x[0] = (x[0] + ks[(i + 1) % 3]).astype(np.uint32)
        x[1] = (x[1] + ks[(i + 2) % 3] + np.uint32(i + 1)).astype(np.uint32)
    return x[0], x[1]


def _random_bits32(key, n):
    b1, b2 = _hash2x32(key[0], key[1],
                       np.zeros(n, np.uint32), np.arange(n, dtype=np.uint32))
    return b1 ^ b2


@functools.lru_cache(maxsize=1)
def _sampled_indices() -> tuple[int, ...]:
    key = np.array([_SEED >> 32, _SEED & 0xFFFFFFFF], dtype=np.uint32)
    b1, b2 = _hash2x32(key[0], key[1],
                       np.zeros(2, np.uint32), np.arange(2, dtype=np.uint32))
    k1, k2 = np.stack([b1, b2], axis=1)
    higher = _random_bits32(k1, _GLOBAL_SIZE)
    lower = _random_bits32(k2, _GLOBAL_SIZE)
    span = np.uint32(_NUM_IMGS)
    mult = np.uint32((((2 ** 16) % int(span)) ** 2) % int(span))
    off = ((higher % span) * mult + (lower % span)).astype(np.uint32)
    off = (off % span).astype(np.int32)
    return tuple(int(v) for v in np.sort(off))


def _tec_body(x_hbm, out_hbm, buf, g0, g1, g2, s0, s1, s2):
    gsem = (g0, g1, g2)
    ssem = (s0, s1, s2)
    cid = lax.axis_index("c")
    sid = lax.axis_index("s")
    w = sid * _NC + cid
    src = jnp.int32(0)
    for i, v in enumerate(_sampled_indices()):
        src = lax.select(w == i, jnp.int32(v), src)
    src_row0 = src * _ROWS
    dst_row0 = w * _ROWS

    def gather(ch, b):
        return pltpu.make_async_copy(
            x_hbm.at[pl.ds(src_row0 + ch * _CHUNK, _CHUNK)],
            buf.at[b], gsem[b])

    def scatter(ch, b):
        return pltpu.make_async_copy(
            buf.at[b],
            out_hbm.at[pl.ds(dst_row0 + ch * _CHUNK, _CHUNK)],
            ssem[b])

    for b in range(_NBUF):
        gather(b, b).start()
    for ch in range(_NCHUNK):
        b = ch % _NBUF
        gather(ch, b).wait()
        scatter(ch, b).start()
        nxt = ch + _NBUF
        if nxt < _NCHUNK:
            scatter(ch, b).wait()
            gather(nxt, b).start()
    for ch in range(_NCHUNK - _NBUF, _NCHUNK):
        scatter(ch, ch % _NBUF).wait()


def kernel(x):
    n, r, c = x.shape
    x2d = x.reshape(n * _ROWS, _COLS)
    out2d = pl.kernel(
        _tec_body,
        out_type=jax.ShapeDtypeStruct((_GLOBAL_SIZE * _ROWS, _COLS), x.dtype),
        mesh=plsc.VectorSubcoreMesh(core_axis_name="c", subcore_axis_name="s"),
        scratch_types=(
            [pltpu.VMEM((_NBUF, _CHUNK, _COLS), jnp.float32)]
            + [pltpu.SemaphoreType.DMA] * (2 * _NBUF)
        ),
    )(x2d)
    return out2d.reshape(_GLOBAL_SIZE, r, c)


# speedup vs baseline: 6.3706x; 1.0004x over previous
import functools

import numpy as np
import jax
from jax import lax
import jax.numpy as jnp
from jax.experimental import pallas as pl
from jax.experimental.pallas import tpu as pltpu
from jax.experimental.pallas import tpu_sc as plsc

_NUM_IMGS = 128
_GLOBAL_SIZE = 32
_SEED = 41

_SLAB = 2048 * 256
_COLS = 256
_ROWS = _SLAB // _COLS
_BIG = 248
_CHUNKS = tuple((i * _BIG, _BIG) for i in range(8)) + ((8 * _BIG, 64),)
_NBUF = 2
_NC = 2

_ROTS = ((13, 15, 26, 6), (17, 29, 16, 24))


def _rotl(x, d):
    return ((x << np.uint32(d)) | (x >> np.uint32(32 - d))).astype(np.uint32)


def _hash2x32(k1, k2, x0, x1):
    x = [x0.astype(np.uint32), x1.astype(np.uint32)]
    ks = [np.uint32(k1), np.uint32(k2),
          np.uint32(k1) ^ np.uint32(k2) ^ np.uint32(0x1BD11BDA)]
    x[0] = (x[0] + ks[0]).astype(np.uint32)
    x[1] = (x[1] + ks[1]).astype(np.uint32)
    for i in range(5):
        for r in _ROTS[i % 2]:
            x[0] = (x[0] + x[1]).astype(np.uint32)
            x[1] = _rotl(x[1], r) ^ x[0]
        x[0] = (x[0] + ks[(i + 1) % 3]).astype(np.uint32)
        x[1] = (x[1] + ks[(i + 2) % 3] + np.uint32(i + 1)).astype(np.uint32)
    return x[0], x[1]


def _random_bits32(key, n):
    b1, b2 = _hash2x32(key[0], key[1],
                       np.zeros(n, np.uint32), np.arange(n, dtype=np.uint32))
    return b1 ^ b2


@functools.lru_cache(maxsize=1)
def _sampled_indices() -> tuple[int, ...]:
    key = np.array([_SEED >> 32, _SEED & 0xFFFFFFFF], dtype=np.uint32)
    b1, b2 = _hash2x32(key[0], key[1],
                       np.zeros(2, np.uint32), np.arange(2, dtype=np.uint32))
    k1, k2 = np.stack([b1, b2], axis=1)
    higher = _random_bits32(k1, _GLOBAL_SIZE)
    lower = _random_bits32(k2, _GLOBAL_SIZE)
    span = np.uint32(_NUM_IMGS)
    mult = np.uint32((((2 ** 16) % int(span)) ** 2) % int(span))
    off = ((higher % span) * mult + (lower % span)).astype(np.uint32)
    off = (off % span).astype(np.int32)
    return tuple(int(v) for v in np.sort(off))


def _tec_body(x_hbm, out_hbm, buf, g0, g1, g2, s0, s1, s2):
    gsem = (g0, g1, g2)
    ssem = (s0, s1, s2)
    cid = lax.axis_index("c")
    sid = lax.axis_index("s")
    w = sid * _NC + cid
    src = jnp.int32(0)
    for i, v in enumerate(_sampled_indices()):
        src = lax.select(w == i, jnp.int32(v), src)
    src_row0 = src * _ROWS
    dst_row0 = w * _ROWS

    nchunk = len(_CHUNKS)

    def gather(ch, b):
        off, rows = _CHUNKS[ch]
        return pltpu.make_async_copy(
            x_hbm.at[pl.ds(src_row0 + off, rows)],
            buf.at[b].at[pl.ds(0, rows)], gsem[b])

    def scatter(ch, b):
        off, rows = _CHUNKS[ch]
        return pltpu.make_async_copy(
            buf.at[b].at[pl.ds(0, rows)],
            out_hbm.at[pl.ds(dst_row0 + off, rows)],
            ssem[b])

    for b in range(_NBUF):
        gather(b, b).start()
    for ch in range(nchunk):
        b = ch % _NBUF
        gather(ch, b).wait()
        scatter(ch, b).start()
        nxt = ch + _NBUF
        if nxt < nchunk:
            scatter(ch, b).wait()
            gather(nxt, b).start()
    for ch in range(nchunk - _NBUF, nchunk):
        scatter(ch, ch % _NBUF).wait()


def kernel(x):
    n, r, c = x.shape
    x2d = x.reshape(n * _ROWS, _COLS)
    out2d = pl.kernel(
        _tec_body,
        out_type=jax.ShapeDtypeStruct((_GLOBAL_SIZE * _ROWS, _COLS), x.dtype),
        mesh=plsc.VectorSubcoreMesh(core_axis_name="c", subcore_axis_name="s"),
        scratch_types=(
            [pltpu.VMEM((_NBUF, _BIG, _COLS), jnp.float32)]
            + [pltpu.SemaphoreType.DMA] * 6
        ),
    )(x2d)
    return out2d.reshape(_GLOBAL_SIZE, r, c)
